# baseline (speedup 1.0000x reference)
import jax
import jax.numpy as jnp
from jax import lax
from jax.experimental import pallas as pl
from jax.experimental.pallas import tpu as pltpu
from jax.experimental.pallas import tpu_sc as plsc

N_NODES = 10000
N_EDGES = 320000
FEAT = 128

NC = 2
NS = 16
NT = NC * NS
CHUNK = 128
N_CHUNK_ROWS = N_EDGES // CHUNK
CHUNKS_PER_TILE = N_CHUNK_ROWS // NT
N_LEFTOVER = N_CHUNK_ROWS - NT * CHUNKS_PER_TILE
N_PAD = 10112
ROWS_PER_TILE = N_PAD // NS
WCHUNK = 80
WTAIL = ROWS_PER_TILE - 7 * WCHUNK


def _sc_body(edges_hbm, feat_hbm, out_hbm, hist_hbm,
             si0, si1, si2, si3, di0, di1, di2, di3, ra, rb, hist_v, acc_sh,
             is0, is1, is2, is3, ga, gb, sa, sb, wsem):
    c = lax.axis_index("c")
    s = lax.axis_index("s")
    t = c * NS + s
    sslot = [si0, si1, si2, si3]
    dslot = [di0, di1, di2, di3]
    isem = [is0, is1, is2, is3]
    rows = [ra, rb]
    gsem = [ga, gb]
    ssem = [sa, sb]
    cb = t * CHUNKS_PER_TILE

    def idx_load(row, k):
        off = pl.multiple_of(row * CHUNK, CHUNK)
        pltpu.async_copy(edges_hbm.at[0, pl.ds(off, CHUNK)], sslot[k],
                         isem[k])
        pltpu.async_copy(edges_hbm.at[1, pl.ds(off, CHUNK)], dslot[k],
                         isem[k])

    def idx_wait(row, k):
        off = pl.multiple_of(row * CHUNK, CHUNK)
        pltpu.make_async_copy(edges_hbm.at[0, pl.ds(off, CHUNK)], sslot[k],
                              isem[k]).wait()
        pltpu.make_async_copy(edges_hbm.at[1, pl.ds(off, CHUNK)], dslot[k],
                              isem[k]).wait()

    def gather(k, b):
        pltpu.async_copy(feat_hbm.at[sslot[k]], rows[b], gsem[b])

    def gather_wait(k, b):
        pltpu.make_async_copy(feat_hbm.at[sslot[k]], rows[b],
                              gsem[b]).wait()

    def scatter(k, b):
        pltpu.async_copy(rows[b], acc_sh.at[dslot[k]], ssem[b], add=True)

    def scatter_wait(k, b):
        pltpu.make_async_copy(rows[b], acc_sh.at[dslot[k]],
                              ssem[b]).wait()

    ones16 = jnp.ones((16,), jnp.float32)

    def hist_update(k):
        for j in range(CHUNK // 16):
            d = dslot[k][pl.ds(j * 16, 16)]
            plsc.addupdate_scatter(hist_v, [d], ones16)

    zvec = jnp.zeros((16,), jnp.float32)

    def zhist(i, carry):
        hist_v[pl.ds(i * 16, 16)] = zvec
        return carry

    lax.fori_loop(0, N_PAD // 16, zhist, 0)

    def zrow(i, carry):
        for j in range(FEAT // 16):
            ra[i, pl.ds(j * 16, 16)] = zvec
        return carry

    lax.fori_loop(0, CHUNK, zrow, 0)
    rbase = s * ROWS_PER_TILE
    zcps = [pltpu.async_copy(ra.at[pl.ds(0, WCHUNK)],
                             acc_sh.at[pl.ds(rbase + k * WCHUNK, WCHUNK)],
                             wsem)
            for k in range(7)]
    zcps.append(pltpu.async_copy(ra.at[pl.ds(0, WTAIL)],
                                 acc_sh.at[pl.ds(rbase + 7 * WCHUNK, WTAIL)],
                                 wsem))
    for cp in zcps:
        cp.wait()
    plsc.subcore_barrier()

    idx_load(cb + 0, 0)
    idx_load(cb + 1, 1)
    idx_wait(cb + 0, 0)
    gather(0, 0)
    for it in range(CHUNKS_PER_TILE):
        k, b = it % 4, it % 2
        if it + 1 < CHUNKS_PER_TILE:
            idx_wait(cb + it + 1, (it + 1) % 4)
            if it >= 1:
                scatter_wait((it - 1) % 4, (it - 1) % 2)
            gather((it + 1) % 4, (it + 1) % 2)
        gather_wait(k, b)
        scatter(k, b)
        hist_update(k)
        if it + 2 < CHUNKS_PER_TILE:
            idx_load(cb + it + 2, (it + 2) % 4)
    for it in (CHUNKS_PER_TILE - 2, CHUNKS_PER_TILE - 1):
        scatter_wait(it % 4, it % 2)

    @pl.when(t < N_LEFTOVER)
    def _():
        row = NT * CHUNKS_PER_TILE + t
        idx_load(row, 0)
        idx_wait(row, 0)
        gather(0, 0)
        gather_wait(0, 0)
        scatter(0, 0)
        hist_update(0)
        scatter_wait(0, 0)

    plsc.subcore_barrier()

    for k in range(7):
        r0_ = rbase + k * WCHUNK
        pltpu.sync_copy(acc_sh.at[pl.ds(r0_, WCHUNK)],
                        ra.at[pl.ds(0, WCHUNK)])
        pltpu.sync_copy(ra.at[pl.ds(0, WCHUNK)],
                        out_hbm.at[c, pl.ds(r0_, WCHUNK)])
    r0_ = rbase + 7 * WCHUNK
    pltpu.sync_copy(acc_sh.at[pl.ds(r0_, WTAIL)], ra.at[pl.ds(0, WTAIL)])
    pltpu.sync_copy(ra.at[pl.ds(0, WTAIL)], out_hbm.at[c, pl.ds(r0_, WTAIL)])
    pltpu.sync_copy(hist_v, hist_hbm.at[c, s])


def _make_sc_scatter():
    mesh = plsc.VectorSubcoreMesh(core_axis_name="c", subcore_axis_name="s")
    return pl.kernel(
        _sc_body,
        mesh=mesh,
        out_type=(
            jax.ShapeDtypeStruct((NC, N_PAD, FEAT), jnp.float32),
            jax.ShapeDtypeStruct((NC, NS, N_PAD), jnp.float32),
        ),
        scratch_types=[
            pltpu.VMEM((CHUNK,), jnp.int32),
            pltpu.VMEM((CHUNK,), jnp.int32),
            pltpu.VMEM((CHUNK,), jnp.int32),
            pltpu.VMEM((CHUNK,), jnp.int32),
            pltpu.VMEM((CHUNK,), jnp.int32),
            pltpu.VMEM((CHUNK,), jnp.int32),
            pltpu.VMEM((CHUNK,), jnp.int32),
            pltpu.VMEM((CHUNK,), jnp.int32),
            pltpu.VMEM((CHUNK, FEAT), jnp.float32),
            pltpu.VMEM((CHUNK, FEAT), jnp.float32),
            pltpu.VMEM((N_PAD,), jnp.float32),
            pltpu.VMEM_SHARED((N_PAD, FEAT), jnp.float32),
        ] + [pltpu.SemaphoreType.DMA for _ in range(9)],
        compiler_params=pltpu.CompilerParams(use_tc_tiling_on_sc=True,
                                             needs_layout_passes=False),
    )


_BLK = 128
_NBLK = N_PAD // _BLK


def _tc_body(self_ref, p0_ref, p1_ref, deg_ref, w_ref, out_ref):
    acc = p0_ref[0] + p1_ref[0]
    d = deg_ref[...]
    lane = jax.lax.broadcasted_iota(jnp.int32, d.shape, 1)
    dcol = jnp.sum(jnp.where(lane == pl.program_id(0), d, 0.0),
                   axis=1, keepdims=True)
    aggn = acc / jnp.maximum(dcol, 1.0)
    out_ref[...] = (
        jnp.dot(self_ref[...], w_ref[:FEAT, :],
                preferred_element_type=jnp.float32)
        + jnp.dot(aggn, w_ref[FEAT:, :], preferred_element_type=jnp.float32))


def _make_tc_combine():
    return pl.pallas_call(
        _tc_body,
        grid=(_NBLK,),
        in_specs=[
            pl.BlockSpec((_BLK, FEAT), lambda i: (i, 0)),
            pl.BlockSpec((1, _BLK, FEAT), lambda i: (0, i, 0)),
            pl.BlockSpec((1, _BLK, FEAT), lambda i: (1, i, 0)),
            pl.BlockSpec((_BLK, _NBLK), lambda i: (0, 0)),
            pl.BlockSpec((2 * FEAT, FEAT), lambda i: (0, 0)),
        ],
        out_specs=pl.BlockSpec((_BLK, FEAT), lambda i: (i, 0)),
        out_shape=jax.ShapeDtypeStruct((N_NODES, FEAT), jnp.float32),
    )


def kernel(x, edge_index, w1, w2):
    sc_scatter = _make_sc_scatter()
    tc_combine = _make_tc_combine()

    p1, hist1 = sc_scatter(edge_index, x)
    deg_t = hist1.sum(axis=(0, 1)).reshape(_NBLK, _BLK).T
    h1 = tc_combine(x, p1, p1, deg_t, w1)
    p2, _ = sc_scatter(edge_index, h1)
    return tc_combine(h1, p2, p2, deg_t, w2)


# speedup vs baseline: 1.2957x; 1.2957x over previous
import jax
import jax.numpy as jnp
from jax import lax
from jax.experimental import pallas as pl
from jax.experimental.pallas import tpu as pltpu
from jax.experimental.pallas import tpu_sc as plsc

N_NODES = 10000
N_EDGES = 320000
FEAT = 128

NC = 2
NS = 16
NT = NC * NS
CHUNK = 128
N_CHUNK_ROWS = N_EDGES // CHUNK
CHUNKS_PER_TILE = N_CHUNK_ROWS // NT
N_LEFTOVER = N_CHUNK_ROWS - NT * CHUNKS_PER_TILE
N_PAD = 10112
ROWS_PER_TILE = N_PAD // NS
WCHUNK = 80
WTAIL = ROWS_PER_TILE - 7 * WCHUNK


def _sc_body(edges_hbm, feat_hbm, out_hbm, hist_hbm,
             si0, si1, si2, si3, di0, di1, di2, di3, ra, rb, hist_v, acc_sh,
             is0, is1, is2, is3, ga, gb, sa, sb, wsem):
    c = lax.axis_index("c")
    s = lax.axis_index("s")
    t = c * NS + s
    sslot = [si0, si1, si2, si3]
    dslot = [di0, di1, di2, di3]
    isem = [is0, is1, is2, is3]
    rows = [ra, rb]
    gsem = [ga, gb]
    ssem = [sa, sb]
    cb = t * CHUNKS_PER_TILE

    def idx_load(row, k):
        off = pl.multiple_of(row * CHUNK, CHUNK)
        pltpu.async_copy(edges_hbm.at[0, pl.ds(off, CHUNK)], sslot[k],
                         isem[k])
        pltpu.async_copy(edges_hbm.at[1, pl.ds(off, CHUNK)], dslot[k],
                         isem[k])

    def idx_wait(row, k):
        off = pl.multiple_of(row * CHUNK, CHUNK)
        pltpu.make_async_copy(edges_hbm.at[0, pl.ds(off, CHUNK)], sslot[k],
                              isem[k]).wait()
        pltpu.make_async_copy(edges_hbm.at[1, pl.ds(off, CHUNK)], dslot[k],
                              isem[k]).wait()

    def gather(k, b):
        pltpu.async_copy(feat_hbm.at[sslot[k]], rows[b], gsem[b])

    def gather_wait(k, b):
        pltpu.make_async_copy(feat_hbm.at[sslot[k]], rows[b],
                              gsem[b]).wait()

    def scatter(k, b):
        pltpu.async_copy(rows[b], acc_sh.at[dslot[k]], ssem[b], add=True)

    def scatter_wait(k, b):
        pltpu.make_async_copy(rows[b], acc_sh.at[dslot[k]],
                              ssem[b]).wait()

    ones16 = jnp.ones((16,), jnp.float32)

    def hist_update(k):
        for j in range(CHUNK // 16):
            d = dslot[k][pl.ds(j * 16, 16)]
            plsc.addupdate_scatter(hist_v, [d], ones16)

    idx_load(cb + 0, 0)
    idx_load(cb + 1, 1)

    zvec = jnp.zeros((16,), jnp.float32)

    def zrow(i, carry):
        for j in range(FEAT // 16):
            rb[i, pl.ds(j * 16, 16)] = zvec
        return carry

    lax.fori_loop(0, CHUNK, zrow, 0)
    rbase = s * ROWS_PER_TILE
    zcps = [pltpu.async_copy(rb.at[pl.ds(0, WCHUNK)],
                             acc_sh.at[pl.ds(rbase + k * WCHUNK, WCHUNK)],
                             wsem)
            for k in range(7)]
    zcps.append(pltpu.async_copy(rb.at[pl.ds(0, WTAIL)],
                                 acc_sh.at[pl.ds(rbase + 7 * WCHUNK, WTAIL)],
                                 wsem))

    def zhist(i, carry):
        hist_v[pl.ds(i * 16, 16)] = zvec
        return carry

    lax.fori_loop(0, N_PAD // 16, zhist, 0)

    idx_wait(cb + 0, 0)
    gather(0, 0)
    for cp in zcps:
        cp.wait()
    plsc.subcore_barrier()

    for it in range(CHUNKS_PER_TILE):
        k, b = it % 4, it % 2
        if it + 1 < CHUNKS_PER_TILE:
            idx_wait(cb + it + 1, (it + 1) % 4)
            if it >= 1:
                scatter_wait((it - 1) % 4, (it - 1) % 2)
            gather((it + 1) % 4, (it + 1) % 2)
        gather_wait(k, b)
        scatter(k, b)
        hist_update(k)
        if it + 2 < CHUNKS_PER_TILE:
            idx_load(cb + it + 2, (it + 2) % 4)
    for it in (CHUNKS_PER_TILE - 2, CHUNKS_PER_TILE - 1):
        scatter_wait(it % 4, it % 2)

    @pl.when(t < N_LEFTOVER)
    def _():
        row = NT * CHUNKS_PER_TILE + t
        idx_load(row, 0)
        idx_wait(row, 0)
        gather(0, 0)
        gather_wait(0, 0)
        scatter(0, 0)
        hist_update(0)
        scatter_wait(0, 0)

    plsc.subcore_barrier()

    hcp = pltpu.async_copy(hist_v, hist_hbm.at[c, s], wsem)
    wcps = []
    for k in range(8):
        buf = rows[k % 2]
        sz = WCHUNK if k < 7 else WTAIL
        r0_ = rbase + k * WCHUNK
        if k >= 2:
            wcps[k - 2].wait()
        pltpu.sync_copy(acc_sh.at[pl.ds(r0_, sz)], buf.at[pl.ds(0, sz)])
        wcps.append(pltpu.async_copy(buf.at[pl.ds(0, sz)],
                                     out_hbm.at[c, pl.ds(r0_, sz)], wsem))
    wcps[-2].wait()
    wcps[-1].wait()
    hcp.wait()


def _make_sc_scatter():
    mesh = plsc.VectorSubcoreMesh(core_axis_name="c", subcore_axis_name="s")
    return pl.kernel(
        _sc_body,
        mesh=mesh,
        out_type=(
            jax.ShapeDtypeStruct((NC, N_PAD, FEAT), jnp.float32),
            jax.ShapeDtypeStruct((NC, NS, N_PAD), jnp.float32),
        ),
        scratch_types=[
            pltpu.VMEM((CHUNK,), jnp.int32),
            pltpu.VMEM((CHUNK,), jnp.int32),
            pltpu.VMEM((CHUNK,), jnp.int32),
            pltpu.VMEM((CHUNK,), jnp.int32),
            pltpu.VMEM((CHUNK,), jnp.int32),
            pltpu.VMEM((CHUNK,), jnp.int32),
            pltpu.VMEM((CHUNK,), jnp.int32),
            pltpu.VMEM((CHUNK,), jnp.int32),
            pltpu.VMEM((CHUNK, FEAT), jnp.float32),
            pltpu.VMEM((CHUNK, FEAT), jnp.float32),
            pltpu.VMEM((N_PAD,), jnp.float32),
            pltpu.VMEM_SHARED((N_PAD, FEAT), jnp.float32),
        ] + [pltpu.SemaphoreType.DMA for _ in range(9)],
        compiler_params=pltpu.CompilerParams(use_tc_tiling_on_sc=True,
                                             needs_layout_passes=False),
    )


_BLK = 1000


def _tc_body(self_ref, p0_ref, p1_ref, deg_ref, w_ref, out_ref):
    acc = p0_ref[0] + p1_ref[0]
    aggn = acc / jnp.maximum(deg_ref[...], 1.0)
    out_ref[...] = (
        jnp.dot(self_ref[...], w_ref[:FEAT, :],
                preferred_element_type=jnp.float32)
        + jnp.dot(aggn, w_ref[FEAT:, :], preferred_element_type=jnp.float32))


def _make_tc_combine():
    return pl.pallas_call(
        _tc_body,
        grid=(N_NODES // _BLK,),
        in_specs=[
            pl.BlockSpec((_BLK, FEAT), lambda i: (i, 0)),
            pl.BlockSpec((1, _BLK, FEAT), lambda i: (0, i, 0)),
            pl.BlockSpec((1, _BLK, FEAT), lambda i: (1, i, 0)),
            pl.BlockSpec((_BLK, 1), lambda i: (i, 0)),
            pl.BlockSpec((2 * FEAT, FEAT), lambda i: (0, 0)),
        ],
        out_specs=pl.BlockSpec((_BLK, FEAT), lambda i: (i, 0)),
        out_shape=jax.ShapeDtypeStruct((N_NODES, FEAT), jnp.float32),
    )


def kernel(x, edge_index, w1, w2):
    sc_scatter = _make_sc_scatter()
    tc_combine = _make_tc_combine()

    p1, hist1 = sc_scatter(edge_index, x)
    deg = hist1.sum(axis=(0, 1)).reshape(N_PAD, 1)
    h1 = tc_combine(x, p1, p1, deg, w1)
    p2, _ = sc_scatter(edge_index, h1)
    return tc_combine(h1, p2, p2, deg, w2)


# speedup vs baseline: 1.3249x; 1.0225x over previous
import jax
import jax.numpy as jnp
from jax import lax
from jax.experimental import pallas as pl
from jax.experimental.pallas import tpu as pltpu
from jax.experimental.pallas import tpu_sc as plsc

N_NODES = 10000
N_EDGES = 320000
FEAT = 128

NC = 2
NS = 16
NT = NC * NS
CHUNK = 128
N_CHUNK_ROWS = N_EDGES // CHUNK
CHUNKS_PER_TILE = N_CHUNK_ROWS // NT
N_LEFTOVER = N_CHUNK_ROWS - NT * CHUNKS_PER_TILE
N_PAD = 10112
ROWS_PER_TILE = N_PAD // NS
WCHUNK = 80
WTAIL = ROWS_PER_TILE - 7 * WCHUNK


def _sc_body(edges_hbm, feat_hbm, out_hbm, hist_hbm,
             si0, si1, si2, si3, di0, di1, di2, di3, ra, rb, hist_v, acc_sh,
             is0, is1, is2, is3, ga, gb, sa, sb, wsem):
    c = lax.axis_index("c")
    s = lax.axis_index("s")
    t = c * NS + s
    sslot = [si0, si1, si2, si3]
    dslot = [di0, di1, di2, di3]
    isem = [is0, is1, is2, is3]
    rows = [ra, rb]
    gsem = [ga, gb]
    ssem = [sa, sb]
    cb = t * CHUNKS_PER_TILE

    def idx_load(row, k):
        off = pl.multiple_of(row * CHUNK, CHUNK)
        pltpu.async_copy(edges_hbm.at[0, pl.ds(off, CHUNK)], sslot[k],
                         isem[k])
        pltpu.async_copy(edges_hbm.at[1, pl.ds(off, CHUNK)], dslot[k],
                         isem[k])

    def idx_wait(row, k):
        off = pl.multiple_of(row * CHUNK, CHUNK)
        pltpu.make_async_copy(edges_hbm.at[0, pl.ds(off, CHUNK)], sslot[k],
                              isem[k]).wait()
        pltpu.make_async_copy(edges_hbm.at[1, pl.ds(off, CHUNK)], dslot[k],
                              isem[k]).wait()

    def gather(k, b):
        pltpu.async_copy(feat_hbm.at[sslot[k]], rows[b], gsem[b])

    def gather_wait(k, b):
        pltpu.make_async_copy(feat_hbm.at[sslot[k]], rows[b],
                              gsem[b]).wait()

    def scatter(k, b):
        pltpu.async_copy(rows[b], acc_sh.at[dslot[k]], ssem[b], add=True)

    def scatter_wait(k, b):
        pltpu.make_async_copy(rows[b], acc_sh.at[dslot[k]],
                              ssem[b]).wait()

    ones16 = jnp.ones((16,), jnp.float32)

    def hist_update(k):
        for j in range(CHUNK // 16):
            d = dslot[k][pl.ds(j * 16, 16)]
            plsc.addupdate_scatter(hist_v, [d], ones16)

    idx_load(cb + 0, 0)
    idx_load(cb + 1, 1)

    zvec = jnp.zeros((16,), jnp.float32)

    def zrow(i, carry):
        for j in range(FEAT // 16):
            rb[i, pl.ds(j * 16, 16)] = zvec
        return carry

    lax.fori_loop(0, CHUNK, zrow, 0)
    rbase = s * ROWS_PER_TILE
    zcps = [pltpu.async_copy(rb.at[pl.ds(0, WCHUNK)],
                             acc_sh.at[pl.ds(rbase + k * WCHUNK, WCHUNK)],
                             wsem)
            for k in range(7)]
    zcps.append(pltpu.async_copy(rb.at[pl.ds(0, WTAIL)],
                                 acc_sh.at[pl.ds(rbase + 7 * WCHUNK, WTAIL)],
                                 wsem))

    def zhist(i, carry):
        hist_v[pl.ds(i * 16, 16)] = zvec
        return carry

    lax.fori_loop(0, N_PAD // 16, zhist, 0)

    idx_wait(cb + 0, 0)
    gather(0, 0)
    for cp in zcps:
        cp.wait()
    plsc.subcore_barrier()

    for it in range(CHUNKS_PER_TILE):
        k, b = it % 4, it % 2
        if it + 1 < CHUNKS_PER_TILE:
            idx_wait(cb + it + 1, (it + 1) % 4)
            if it >= 1:
                scatter_wait((it - 1) % 4, (it - 1) % 2)
            gather((it + 1) % 4, (it + 1) % 2)
        gather_wait(k, b)
        scatter(k, b)
        hist_update(k)
        if it + 2 < CHUNKS_PER_TILE:
            idx_load(cb + it + 2, (it + 2) % 4)
    for it in (CHUNKS_PER_TILE - 2, CHUNKS_PER_TILE - 1):
        scatter_wait(it % 4, it % 2)

    @pl.when(t < N_LEFTOVER)
    def _():
        row = NT * CHUNKS_PER_TILE + t
        idx_load(row, 0)
        idx_wait(row, 0)
        gather(0, 0)
        gather_wait(0, 0)
        scatter(0, 0)
        hist_update(0)
        scatter_wait(0, 0)

    plsc.subcore_barrier()

    hcp = pltpu.async_copy(hist_v, hist_hbm.at[c, s], wsem)
    wcps = []
    for k in range(8):
        buf = rows[k % 2]
        sz = WCHUNK if k < 7 else WTAIL
        r0_ = rbase + k * WCHUNK
        if k >= 2:
            wcps[k - 2].wait()
        pltpu.sync_copy(acc_sh.at[pl.ds(r0_, sz)], buf.at[pl.ds(0, sz)])
        wcps.append(pltpu.async_copy(buf.at[pl.ds(0, sz)],
                                     out_hbm.at[c, pl.ds(r0_, sz)], wsem))
    wcps[-2].wait()
    wcps[-1].wait()
    hcp.wait()


def _make_sc_scatter():
    mesh = plsc.VectorSubcoreMesh(core_axis_name="c", subcore_axis_name="s")
    return pl.kernel(
        _sc_body,
        mesh=mesh,
        out_type=(
            jax.ShapeDtypeStruct((NC, N_PAD, FEAT), jnp.float32),
            jax.ShapeDtypeStruct((NC, NS, N_PAD), jnp.float32),
        ),
        scratch_types=[
            pltpu.VMEM((CHUNK,), jnp.int32),
            pltpu.VMEM((CHUNK,), jnp.int32),
            pltpu.VMEM((CHUNK,), jnp.int32),
            pltpu.VMEM((CHUNK,), jnp.int32),
            pltpu.VMEM((CHUNK,), jnp.int32),
            pltpu.VMEM((CHUNK,), jnp.int32),
            pltpu.VMEM((CHUNK,), jnp.int32),
            pltpu.VMEM((CHUNK,), jnp.int32),
            pltpu.VMEM((CHUNK, FEAT), jnp.float32),
            pltpu.VMEM((CHUNK, FEAT), jnp.float32),
            pltpu.VMEM((N_PAD,), jnp.float32),
            pltpu.VMEM_SHARED((N_PAD, FEAT), jnp.float32),
        ] + [pltpu.SemaphoreType.DMA for _ in range(9)],
        compiler_params=pltpu.CompilerParams(use_tc_tiling_on_sc=True,
                                             needs_layout_passes=False),
    )


_BLK = 2000


def _tc_self_body(self_ref, w_ref, out_ref):
    out_ref[...] = jnp.dot(self_ref[...], w_ref[:FEAT, :],
                           preferred_element_type=jnp.float32)


def _make_tc_self():
    return pl.pallas_call(
        _tc_self_body,
        grid=(N_NODES // _BLK,),
        in_specs=[
            pl.BlockSpec((_BLK, FEAT), lambda i: (i, 0)),
            pl.BlockSpec((2 * FEAT, FEAT), lambda i: (0, 0)),
        ],
        out_specs=pl.BlockSpec((_BLK, FEAT), lambda i: (i, 0)),
        out_shape=jax.ShapeDtypeStruct((N_NODES, FEAT), jnp.float32),
    )


def _tc_comb_body(sp_ref, p0_ref, p1_ref, deg_ref, w_ref, out_ref):
    acc = p0_ref[0] + p1_ref[0]
    aggn = acc / jnp.maximum(deg_ref[...], 1.0)
    out_ref[...] = sp_ref[...] + jnp.dot(aggn, w_ref[FEAT:, :],
                                         preferred_element_type=jnp.float32)


def _make_tc_combine():
    return pl.pallas_call(
        _tc_comb_body,
        grid=(N_NODES // _BLK,),
        in_specs=[
            pl.BlockSpec((_BLK, FEAT), lambda i: (i, 0)),
            pl.BlockSpec((1, _BLK, FEAT), lambda i: (0, i, 0)),
            pl.BlockSpec((1, _BLK, FEAT), lambda i: (1, i, 0)),
            pl.BlockSpec((_BLK, 1), lambda i: (i, 0)),
            pl.BlockSpec((2 * FEAT, FEAT), lambda i: (0, 0)),
        ],
        out_specs=pl.BlockSpec((_BLK, FEAT), lambda i: (i, 0)),
        out_shape=jax.ShapeDtypeStruct((N_NODES, FEAT), jnp.float32),
    )


def kernel(x, edge_index, w1, w2):
    sc_scatter = _make_sc_scatter()
    tc_self = _make_tc_self()
    tc_combine = _make_tc_combine()

    p1, hist1 = sc_scatter(edge_index, x)
    s1 = tc_self(x, w1)
    deg = hist1.sum(axis=(0, 1)).reshape(N_PAD, 1)
    h1 = tc_combine(s1, p1, p1, deg, w1)
    p2, _ = sc_scatter(edge_index, h1)
    s2 = tc_self(h1, w2)
    return tc_combine(s2, p2, p2, deg, w2)


# speedup vs baseline: 1.3553x; 1.0229x over previous
import jax
import jax.numpy as jnp
from jax import lax
from jax.experimental import pallas as pl
from jax.experimental.pallas import tpu as pltpu
from jax.experimental.pallas import tpu_sc as plsc

N_NODES = 10000
N_EDGES = 320000
FEAT = 128

NC = 2
NS = 16
NT = NC * NS
CHUNK = 128
N_CHUNK_ROWS = N_EDGES // CHUNK
CHUNKS_PER_TILE = N_CHUNK_ROWS // NT
N_LEFTOVER = N_CHUNK_ROWS - NT * CHUNKS_PER_TILE
N_PAD = 10112
ROWS_PER_TILE = N_PAD // NS
WCHUNK = 80
WTAIL = ROWS_PER_TILE - 7 * WCHUNK


def _sc_body(edges_hbm, feat_hbm, out_hbm, hist_hbm,
             si0, si1, si2, si3, di0, di1, di2, di3, ra, rb, hist_v, acc_sh,
             is0, is1, is2, is3, ga, gb, sa, sb, wsem):
    c = lax.axis_index("c")
    s = lax.axis_index("s")
    t = c * NS + s
    sslot = [si0, si1, si2, si3]
    dslot = [di0, di1, di2, di3]
    isem = [is0, is1, is2, is3]
    rows = [ra, rb]
    gsem = [ga, gb]
    ssem = [sa, sb]
    cb = t * CHUNKS_PER_TILE

    def idx_load(row, k):
        off = pl.multiple_of(row * CHUNK, CHUNK)
        pltpu.async_copy(edges_hbm.at[0, pl.ds(off, CHUNK)], sslot[k],
                         isem[k])
        pltpu.async_copy(edges_hbm.at[1, pl.ds(off, CHUNK)], dslot[k],
                         isem[k])

    def idx_wait(row, k):
        off = pl.multiple_of(row * CHUNK, CHUNK)
        pltpu.make_async_copy(edges_hbm.at[0, pl.ds(off, CHUNK)], sslot[k],
                              isem[k]).wait()
        pltpu.make_async_copy(edges_hbm.at[1, pl.ds(off, CHUNK)], dslot[k],
                              isem[k]).wait()

    def gather(k, b):
        pltpu.async_copy(feat_hbm.at[sslot[k]], rows[b], gsem[b])

    def gather_wait(k, b):
        pltpu.make_async_copy(feat_hbm.at[sslot[k]], rows[b],
                              gsem[b]).wait()

    def scatter(k, b):
        pltpu.async_copy(rows[b], acc_sh.at[dslot[k]], ssem[b], add=True)

    def scatter_wait(k, b):
        pltpu.make_async_copy(rows[b], acc_sh.at[dslot[k]],
                              ssem[b]).wait()

    ones16 = jnp.ones((16,), jnp.float32)

    def hist_update(k):
        for j in range(CHUNK // 16):
            d = dslot[k][pl.ds(j * 16, 16)]
            plsc.addupdate_scatter(hist_v, [d], ones16)

    idx_load(cb + 0, 0)
    idx_load(cb + 1, 1)

    zvec = jnp.zeros((16,), jnp.float32)

    def zrow(i, carry):
        for j in range(FEAT // 16):
            rb[i, pl.ds(j * 16, 16)] = zvec
        return carry

    lax.fori_loop(0, CHUNK, zrow, 0)
    rbase = s * ROWS_PER_TILE
    zcps = [pltpu.async_copy(rb.at[pl.ds(0, WCHUNK)],
                             acc_sh.at[pl.ds(rbase + k * WCHUNK, WCHUNK)],
                             wsem)
            for k in range(7)]
    zcps.append(pltpu.async_copy(rb.at[pl.ds(0, WTAIL)],
                                 acc_sh.at[pl.ds(rbase + 7 * WCHUNK, WTAIL)],
                                 wsem))

    def zhist(i, carry):
        hist_v[pl.ds(i * 16, 16)] = zvec
        return carry

    lax.fori_loop(0, N_PAD // 16, zhist, 0)

    idx_wait(cb + 0, 0)
    gather(0, 0)
    for cp in zcps:
        cp.wait()
    plsc.subcore_barrier()

    for it in range(CHUNKS_PER_TILE):
        k, b = it % 4, it % 2
        if it + 1 < CHUNKS_PER_TILE:
            idx_wait(cb + it + 1, (it + 1) % 4)
            if it >= 1:
                scatter_wait((it - 1) % 4, (it - 1) % 2)
            gather((it + 1) % 4, (it + 1) % 2)
        gather_wait(k, b)
        scatter(k, b)
        hist_update(k)
        if it + 2 < CHUNKS_PER_TILE:
            idx_load(cb + it + 2, (it + 2) % 4)
    for it in (CHUNKS_PER_TILE - 2, CHUNKS_PER_TILE - 1):
        scatter_wait(it % 4, it % 2)

    @pl.when(t < N_LEFTOVER)
    def _():
        row = NT * CHUNKS_PER_TILE + t
        idx_load(row, 0)
        idx_wait(row, 0)
        gather(0, 0)
        gather_wait(0, 0)
        scatter(0, 0)
        hist_update(0)
        scatter_wait(0, 0)

    plsc.subcore_barrier()

    hcp = pltpu.async_copy(hist_v, hist_hbm.at[c, s], wsem)
    wcps = []
    for k in range(8):
        buf = rows[k % 2]
        sz = WCHUNK if k < 7 else WTAIL
        r0_ = rbase + k * WCHUNK
        if k >= 2:
            wcps[k - 2].wait()
        pltpu.sync_copy(acc_sh.at[pl.ds(r0_, sz)], buf.at[pl.ds(0, sz)])
        wcps.append(pltpu.async_copy(buf.at[pl.ds(0, sz)],
                                     out_hbm.at[c, pl.ds(r0_, sz)], wsem))
    wcps[-2].wait()
    wcps[-1].wait()
    hcp.wait()


def _make_sc_scatter():
    mesh = plsc.VectorSubcoreMesh(core_axis_name="c", subcore_axis_name="s")
    return pl.kernel(
        _sc_body,
        mesh=mesh,
        out_type=(
            jax.ShapeDtypeStruct((NC, N_PAD, FEAT), jnp.float32),
            jax.ShapeDtypeStruct((NC, NS, N_PAD), jnp.float32),
        ),
        scratch_types=[
            pltpu.VMEM((CHUNK,), jnp.int32),
            pltpu.VMEM((CHUNK,), jnp.int32),
            pltpu.VMEM((CHUNK,), jnp.int32),
            pltpu.VMEM((CHUNK,), jnp.int32),
            pltpu.VMEM((CHUNK,), jnp.int32),
            pltpu.VMEM((CHUNK,), jnp.int32),
            pltpu.VMEM((CHUNK,), jnp.int32),
            pltpu.VMEM((CHUNK,), jnp.int32),
            pltpu.VMEM((CHUNK, FEAT), jnp.float32),
            pltpu.VMEM((CHUNK, FEAT), jnp.float32),
            pltpu.VMEM((N_PAD,), jnp.float32),
            pltpu.VMEM_SHARED((N_PAD, FEAT), jnp.float32),
        ] + [pltpu.SemaphoreType.DMA for _ in range(9)],
        compiler_params=pltpu.CompilerParams(use_tc_tiling_on_sc=True,
                                             needs_layout_passes=False),
    )


_BLK = 2048


def _tc_self_body(self_ref, w_ref, out_ref):
    out_ref[...] = jnp.dot(self_ref[...], w_ref[:FEAT, :],
                           preferred_element_type=jnp.float32)


def _make_tc_self():
    return pl.pallas_call(
        _tc_self_body,
        grid=(-(-N_NODES // _BLK),),
        in_specs=[
            pl.BlockSpec((_BLK, FEAT), lambda i: (i, 0)),
            pl.BlockSpec((2 * FEAT, FEAT), lambda i: (0, 0)),
        ],
        out_specs=pl.BlockSpec((_BLK, FEAT), lambda i: (i, 0)),
        out_shape=jax.ShapeDtypeStruct((N_NODES, FEAT), jnp.float32),
    )


def _tc_comb_body(sp_ref, p0_ref, p1_ref, hist_ref, w_ref, out_ref):
    acc = p0_ref[0] + p1_ref[0]
    deg = jnp.maximum(hist_ref[...].sum(axis=(0, 1)), 1.0)
    dcol = jnp.transpose(deg[None, :])
    aggn = acc / dcol
    out_ref[...] = sp_ref[...] + jnp.dot(aggn, w_ref[FEAT:, :],
                                         preferred_element_type=jnp.float32)


def _make_tc_combine():
    return pl.pallas_call(
        _tc_comb_body,
        grid=(-(-N_NODES // _BLK),),
        in_specs=[
            pl.BlockSpec((_BLK, FEAT), lambda i: (i, 0)),
            pl.BlockSpec((1, _BLK, FEAT), lambda i: (0, i, 0)),
            pl.BlockSpec((1, _BLK, FEAT), lambda i: (1, i, 0)),
            pl.BlockSpec((NC, NS, _BLK), lambda i: (0, 0, i)),
            pl.BlockSpec((2 * FEAT, FEAT), lambda i: (0, 0)),
        ],
        out_specs=pl.BlockSpec((_BLK, FEAT), lambda i: (i, 0)),
        out_shape=jax.ShapeDtypeStruct((N_NODES, FEAT), jnp.float32),
    )


def kernel(x, edge_index, w1, w2):
    sc_scatter = _make_sc_scatter()
    tc_self = _make_tc_self()
    tc_combine = _make_tc_combine()

    p1, hist1 = sc_scatter(edge_index, x)
    s1 = tc_self(x, w1)
    h1 = tc_combine(s1, p1, p1, hist1, w1)
    p2, _ = sc_scatter(edge_index, h1)
    s2 = tc_self(h1, w2)
    return tc_combine(s2, p2, p2, hist1, w2)
